# baseline (speedup 1.0000x reference)
import functools

import jax
import jax.numpy as jnp
from jax import lax
from jax.experimental import pallas as pl
from jax.experimental.pallas import tpu as pltpu
from jax.experimental.pallas import tpu_sc as plsc

_NC = 2
_NS = 16
_NW = _NC * _NS
_L = 16


@functools.lru_cache(maxsize=None)
def _build(n_rows, n_cols, dtype_name):
    dtype = jnp.dtype(dtype_name)
    chunk = 64
    rows_per_w = n_rows // _NW
    n_chunks = rows_per_w // chunk
    n_jb = n_cols // _L
    chunk_elems = chunk * n_cols

    mesh = plsc.VectorSubcoreMesh(core_axis_name="c", subcore_axis_name="s")

    @functools.partial(
        pl.kernel,
        out_type=jax.ShapeDtypeStruct((n_rows * n_cols,), dtype),
        mesh=mesh,
        compiler_params=pltpu.CompilerParams(
            use_tc_tiling_on_sc=False, needs_layout_passes=False
        ),
        scratch_types=[
            pltpu.VMEM((n_cols,), jnp.int32),
            pltpu.VMEM((chunk_elems,), dtype),
            pltpu.VMEM((chunk_elems,), dtype),
        ],
    )
    def reorder(x_hbm, attr_hbm, out_hbm, attr_v, in_v, res_v):
        wid = lax.axis_index("s") * _NC + lax.axis_index("c")
        elem0 = wid * rows_per_w * n_cols
        pltpu.sync_copy(attr_hbm, attr_v)
        idx_vecs = [attr_v[pl.ds(jb * _L, _L)] for jb in range(n_jb)]
        lane = jnp.arange(_L, dtype=jnp.int32)

        @pl.loop(0, n_chunks)
        def _chunk_loop(ci):
            base = elem0 + ci * chunk_elems
            pltpu.sync_copy(x_hbm.at[pl.ds(base, chunk_elems)], in_v)

            @pl.loop(0, chunk)
            def _row_loop(r):
                roff = jnp.full((_L,), r * n_cols, dtype=jnp.int32)
                for jb in range(n_jb):
                    vals = plsc.load_gather(in_v, [roff + idx_vecs[jb]])
                    plsc.store_scatter(res_v, [roff + (jb * _L) + lane], vals)

            pltpu.sync_copy(res_v, out_hbm.at[pl.ds(base, chunk_elems)])

    return reorder


def kernel(x, attribution):
    n_rows, n_cols = x.shape
    fn = _build(n_rows, n_cols, str(x.dtype))
    out = fn(x.reshape(-1), attribution.astype(jnp.int32))
    return out.reshape(n_rows, n_cols)
